# baseline (speedup 1.0000x reference)
import functools

import jax
import jax.numpy as jnp
from jax import lax
from jax.experimental import pallas as pl
from jax.experimental.pallas import tpu as pltpu
from jax.experimental.pallas import tpu_sc as plsc

B = 4096
SEQ = 200
D = 128
VOCAB = 100000
CPAD = 16

CH0 = 104
CH1 = SEQ - CH0

NC, NS = 2, 16
NW = NC * NS
BPW = B // NW

_mesh = plsc.VectorSubcoreMesh(core_axis_name="c", subcore_axis_name="s")


@functools.partial(
    pl.kernel,
    out_type=jax.ShapeDtypeStruct((B, CPAD), jnp.float32),
    mesh=_mesh,
    scratch_types=[
        pltpu.VMEM((BPW, SEQ), jnp.int32),
        pltpu.VMEM((SEQ, CPAD), jnp.float32),
        pltpu.VMEM((SEQ, CPAD), jnp.float32),
        pltpu.VMEM((SEQ, CPAD), jnp.float32),
        pltpu.VMEM((BPW, CPAD), jnp.float32),
        pltpu.VMEM((CPAD,), jnp.float32),
        pltpu.SemaphoreType.DMA,
        pltpu.SemaphoreType.DMA,
        pltpu.SemaphoreType.DMA,
    ],
    compiler_params=pltpu.CompilerParams(
        use_tc_tiling_on_sc=False, needs_layout_passes=False
    ),
)
def _pool_sc(x_hbm, tw_hbm, bias_hbm, out_hbm, idx_v, buf0, buf1, buf2,
             out_v, bias_v, sem0, sem1, sem2):
    wid = lax.axis_index("s") * NC + lax.axis_index("c")
    base = wid * BPW

    bufs = (buf0, buf1, buf2)
    sems = (sem0, sem1, sem2)
    nbuf = len(bufs)

    def fire(b, slot):
        buf, sem = bufs[slot], sems[slot]
        pltpu.async_copy(
            tw_hbm.at[idx_v.at[b, pl.ds(0, CH0)]], buf.at[pl.ds(0, CH0)], sem
        )
        pltpu.async_copy(
            tw_hbm.at[idx_v.at[b, pl.ds(CH0, CH1)]], buf.at[pl.ds(CH0, CH1)], sem
        )

    def drain(slot):
        buf, sem = bufs[slot], sems[slot]
        pltpu.make_async_copy(tw_hbm.at[pl.ds(0, SEQ)], buf, sem).wait()

    def accumulate(b, slot, bias):
        buf = bufs[slot]
        zero = jnp.zeros((16,), jnp.float32)

        def body(i, acc):
            r = i * 4
            return tuple(
                acc[j] + buf[r + j, pl.ds(0, 16)] for j in range(4)
            )

        a0, a1, a2, a3 = lax.fori_loop(0, SEQ // 4, body, (zero,) * 4)
        total = (a0 + a1) + (a2 + a3)
        out_v[b, pl.ds(0, 16)] = total * jnp.float32(1.0 / SEQ) + bias

    pltpu.sync_copy(bias_hbm, bias_v)
    bias = bias_v[pl.ds(0, 16)]

    half = BPW // 2
    pltpu.sync_copy(x_hbm.at[pl.ds(base, half)], idx_v.at[pl.ds(0, half)])
    for s in range(nbuf):
        fire(s, s)
    pltpu.sync_copy(
        x_hbm.at[pl.ds(base + half, half)], idx_v.at[pl.ds(half, half)]
    )

    main_iters = BPW // nbuf - 1

    @pl.loop(0, main_iters)
    def _(g):
        b0 = g * nbuf
        for s in range(nbuf):
            drain(s)
            accumulate(b0 + s, s, bias)
            fire(b0 + s + nbuf, s)

    for b in range(main_iters * nbuf, BPW):
        s = b % nbuf
        drain(s)
        accumulate(b, s, bias)
        if b + nbuf < BPW:
            fire(b + nbuf, s)

    pltpu.sync_copy(out_v, out_hbm.at[pl.ds(base, BPW)])


VB = 10000


def _tw_body(t_ref, w_ref, o_ref):
    o_ref[...] = jnp.dot(
        t_ref[...],
        w_ref[...],
        preferred_element_type=jnp.float32,
        precision=jax.lax.Precision.HIGHEST,
    )


_tw = pl.pallas_call(
    _tw_body,
    grid=(VOCAB // VB,),
    in_specs=[
        pl.BlockSpec((VB, D), lambda i: (i, 0)),
        pl.BlockSpec((D, CPAD), lambda i: (0, 0)),
    ],
    out_specs=pl.BlockSpec((VB, CPAD), lambda i: (i, 0)),
    out_shape=jax.ShapeDtypeStruct((VOCAB, CPAD), jnp.float32),
)


def kernel(x, table, W, b):
    wt = jnp.pad(W.T, ((0, 0), (0, CPAD - W.shape[0])))
    tw = _tw(table, wt)
    bp = jnp.pad(b, (0, CPAD - b.shape[0]))
    if x.dtype != jnp.int32:
        x = x.astype(jnp.int32)
    out = _pool_sc(x, tw, bp)
    return out[:, : W.shape[0]]


# speedup vs baseline: 1.0901x; 1.0901x over previous
import functools

import jax
import jax.numpy as jnp
from jax import lax
from jax.experimental import pallas as pl
from jax.experimental.pallas import tpu as pltpu
from jax.experimental.pallas import tpu_sc as plsc

B = 4096
SEQ = 200
D = 128
VOCAB = 100000
CPAD = 16

CH0 = 104
CH1 = SEQ - CH0

NC, NS = 2, 16
NW = NC * NS
BPW = B // NW

_mesh = plsc.VectorSubcoreMesh(core_axis_name="c", subcore_axis_name="s")


@functools.partial(
    pl.kernel,
    out_type=jax.ShapeDtypeStruct((B, CPAD), jnp.float32),
    mesh=_mesh,
    scratch_types=[
        pltpu.VMEM((BPW, SEQ), jnp.int32),
        pltpu.VMEM((SEQ, CPAD), jnp.float32),
        pltpu.VMEM((SEQ, CPAD), jnp.float32),
        pltpu.VMEM((SEQ, CPAD), jnp.float32),
        pltpu.VMEM((BPW, CPAD), jnp.float32),
        pltpu.VMEM((CPAD,), jnp.float32),
        pltpu.SemaphoreType.DMA,
        pltpu.SemaphoreType.DMA,
        pltpu.SemaphoreType.DMA,
    ],
    compiler_params=pltpu.CompilerParams(
        use_tc_tiling_on_sc=False, needs_layout_passes=False
    ),
)
def _pool_sc(x_hbm, tw_hbm, bias_hbm, out_hbm, idx_v, buf0, buf1, buf2,
             out_v, bias_v, sem0, sem1, sem2):
    wid = lax.axis_index("s") * NC + lax.axis_index("c")
    base = wid * BPW

    bufs = (buf0, buf1, buf2)
    sems = (sem0, sem1, sem2)
    nbuf = len(bufs)

    def fire(b, slot):
        buf, sem = bufs[slot], sems[slot]
        pltpu.async_copy(
            tw_hbm.at[idx_v.at[b, pl.ds(0, CH0)]], buf.at[pl.ds(0, CH0)], sem
        )
        pltpu.async_copy(
            tw_hbm.at[idx_v.at[b, pl.ds(CH0, CH1)]], buf.at[pl.ds(CH0, CH1)], sem
        )

    def drain(slot):
        buf, sem = bufs[slot], sems[slot]
        pltpu.make_async_copy(tw_hbm.at[pl.ds(0, SEQ)], buf, sem).wait()

    def accumulate(b, slot, bias):
        buf = bufs[slot]
        zero = jnp.zeros((16,), jnp.float32)

        def body(i, acc):
            r = i * 4
            return tuple(
                acc[j] + buf[r + j, pl.ds(0, 16)] for j in range(4)
            )

        a0, a1, a2, a3 = lax.fori_loop(0, SEQ // 4, body, (zero,) * 4)
        total = (a0 + a1) + (a2 + a3)
        out_v[b, pl.ds(0, 16)] = total * jnp.float32(1.0 / SEQ) + bias

    pltpu.sync_copy(bias_hbm, bias_v)
    bias = bias_v[pl.ds(0, 16)]

    half = BPW // 2
    pltpu.sync_copy(x_hbm.at[pl.ds(base, half)], idx_v.at[pl.ds(0, half)])
    for s in range(nbuf):
        fire(s, s)
    pltpu.sync_copy(
        x_hbm.at[pl.ds(base + half, half)], idx_v.at[pl.ds(half, half)]
    )

    main_iters = BPW // nbuf - 1

    @pl.loop(0, main_iters)
    def _(g):
        b0 = g * nbuf
        for s in range(nbuf):
            drain(s)
            accumulate(b0 + s, s, bias)
            fire(b0 + s + nbuf, s)

    for b in range(main_iters * nbuf, BPW):
        s = b % nbuf
        drain(s)
        accumulate(b, s, bias)
        if b + nbuf < BPW:
            fire(b + nbuf, s)

    pltpu.sync_copy(out_v, out_hbm.at[pl.ds(base, BPW)])


VB = 10000


def _tw_body(t_ref, w_ref, o_ref):
    o_ref[...] = jnp.dot(
        t_ref[...], w_ref[...], preferred_element_type=jnp.float32
    )


_tw = pl.pallas_call(
    _tw_body,
    grid=(VOCAB // VB,),
    in_specs=[
        pl.BlockSpec((VB, D), lambda i: (i, 0)),
        pl.BlockSpec((D, CPAD), lambda i: (0, 0)),
    ],
    out_specs=pl.BlockSpec((VB, CPAD), lambda i: (i, 0)),
    out_shape=jax.ShapeDtypeStruct((VOCAB, CPAD), jnp.float32),
)


def kernel(x, table, W, b):
    wt = jnp.pad(W.T, ((0, 0), (0, CPAD - W.shape[0])))
    tw = _tw(table, wt)
    bp = jnp.pad(b, (0, CPAD - b.shape[0]))
    if x.dtype != jnp.int32:
        x = x.astype(jnp.int32)
    out = _pool_sc(x, tw, bp)
    return out[:, : W.shape[0]]
